# baseline (speedup 1.0000x reference)
import functools

import jax
import jax.numpy as jnp
from jax import lax
from jax.experimental import pallas as pl
from jax.experimental.pallas import tpu as pltpu
from jax.experimental.pallas import tpu_sc as plsc

_B, _NCAT, _NCONT, _V, _D = 16384, 26, 13, 100000, 32

_NROWS = _NCAT * _D
_NW = 32
_ROWS_PER_W = _NROWS // _NW
_CHUNK = 8192
_NCHUNK = _B // _CHUNK
_HALF = 49920
_REST = _V - _HALF


def _gather_body(idx_hbm, tab_hbm, out_hbm, idx_v, buf_a, buf_b, out_v,
                 sem_a, sem_b):
    wid = lax.axis_index("s") * 2 + lax.axis_index("c")
    start = wid * _ROWS_PER_W
    end = start + _ROWS_PER_W
    mid = jnp.minimum((lax.div(start, _D) + 1) * _D, end)
    lanes = lax.iota(jnp.int32, 16)

    def dma_half(r, h, buf, sem):
        sz = _REST if h else _HALF
        pltpu.async_copy(tab_hbm.at[r].at[pl.ds(h * _HALF, sz)], buf, sem)

    def wait_half(h, buf, sem):
        sz = _REST if h else _HALF
        pltpu.make_async_copy(tab_hbm.at[0].at[pl.ds(h * _HALF, sz)], buf,
                              sem).wait()

    def gather_pass(p, buf, c):
        base = p * _HALF
        span = _REST if p else _HALF

        @plsc.parallel_loop(0, _CHUNK, step=16, unroll=8)
        def _g(k):
            i16 = idx_v[pl.ds(c * _CHUNK + k, 16)]
            ia = i16 - base
            m = ia.astype(jnp.uint32) < jnp.uint32(span)
            g = plsc.load_gather(buf, [ia], mask=m)
            plsc.store_scatter(out_v, [lanes + k], g, mask=m)

    def process_row(r, carry):
        rn = jnp.minimum(r + 1, _NROWS - 1)
        wait_half(0, buf_a, sem_a)
        gather_pass(0, buf_a, 0)
        wait_half(1, buf_b, sem_b)
        gather_pass(1, buf_b, 0)
        pltpu.sync_copy(out_v, out_hbm.at[r, pl.ds(0, _CHUNK)])
        gather_pass(0, buf_a, 1)
        dma_half(rn, 0, buf_a, sem_a)
        gather_pass(1, buf_b, 1)
        dma_half(rn, 1, buf_b, sem_b)
        pltpu.sync_copy(out_v, out_hbm.at[r, pl.ds(_CHUNK, _CHUNK)])
        return carry

    pltpu.sync_copy(idx_hbm.at[lax.div(start, _D)], idx_v)
    dma_half(start, 0, buf_a, sem_a)
    dma_half(start, 1, buf_b, sem_b)
    lax.fori_loop(start, mid, process_row, 0)

    @pl.when(mid < end)
    def _second_field():
        pltpu.sync_copy(idx_hbm.at[lax.div(mid, _D)], idx_v)

    lax.fori_loop(mid, end, process_row, 0)
    wait_half(0, buf_a, sem_a)
    wait_half(1, buf_b, sem_b)


def _sc_gather(idxT, tabT):
    mesh = plsc.VectorSubcoreMesh(core_axis_name="c", subcore_axis_name="s")
    k = pl.kernel(
        _gather_body,
        mesh=mesh,
        out_type=jax.ShapeDtypeStruct((_NROWS, _B), jnp.float32),
        scratch_types=[
            pltpu.VMEM((_B,), jnp.int32),
            pltpu.VMEM((_HALF,), jnp.float32),
            pltpu.VMEM((_REST,), jnp.float32),
            pltpu.VMEM((_CHUNK,), jnp.float32),
            pltpu.SemaphoreType.DMA,
            pltpu.SemaphoreType.DMA,
        ],
        compiler_params=pltpu.CompilerParams(needs_layout_passes=False),
    )
    return k(idxT, tabT)


_BT = 2048


def _mlp_body(cat_ref, x_ref, w1a_ref, w1c_ref, b1_ref, w2_ref, b2_ref,
              w3_ref, b3_ref, out_ref):
    x = x_ref[...]
    xc = jnp.where(jnp.isnan(x), 0.0, x)
    x1 = jnp.dot(w1a_ref[...], cat_ref[...], preferred_element_type=jnp.float32)
    x1 = x1 + jnp.dot(w1c_ref[...], xc, preferred_element_type=jnp.float32)
    x1 = jnp.maximum(x1 + b1_ref[...], 0.0)
    x2 = jnp.maximum(
        jnp.dot(w2_ref[...], x1, preferred_element_type=jnp.float32)
        + b2_ref[...], 0.0)
    out_ref[...] = (
        jnp.dot(w3_ref[...], x2, preferred_element_type=jnp.float32)
        + b3_ref[...])


def _mlp(catT, xT, w1a, w1c, b1, w2, b2, w3, b3):
    h1, h2 = w1a.shape[0], w2.shape[0]
    return pl.pallas_call(
        _mlp_body,
        grid=(_B // _BT,),
        in_specs=[
            pl.BlockSpec((_NROWS, _BT), lambda i: (0, i)),
            pl.BlockSpec((_NCONT, _BT), lambda i: (0, i)),
            pl.BlockSpec((h1, _NROWS), lambda i: (0, 0)),
            pl.BlockSpec((h1, _NCONT), lambda i: (0, 0)),
            pl.BlockSpec((h1, 1), lambda i: (0, 0)),
            pl.BlockSpec((h2, h1), lambda i: (0, 0)),
            pl.BlockSpec((h2, 1), lambda i: (0, 0)),
            pl.BlockSpec((1, h2), lambda i: (0, 0)),
            pl.BlockSpec((1, 1), lambda i: (0, 0)),
        ],
        out_specs=pl.BlockSpec((1, _BT), lambda i: (0, i)),
        out_shape=jax.ShapeDtypeStruct((1, _B), jnp.float32),
    )(catT, xT, w1a, w1c, b1, w2, b2, w3, b3)


def kernel(x_cat, x_cont, cat_tables, cont_tables, W1, b1, W2, b2, Wout, bout):
    idxT = x_cat.T
    tabT = cat_tables.transpose(0, 2, 1).reshape(_NROWS, _V)
    catT = _sc_gather(idxT, tabT)

    xT = x_cont.T
    w1a = W1[:, :_NROWS]
    w1c = jnp.einsum("id,jid->ji", cont_tables[:, 1, :],
                     W1[:, _NROWS:].reshape(-1, _NCONT, _D))
    out = _mlp(catT, xT, w1a, w1c, b1.reshape(-1, 1), W2,
               b2.reshape(-1, 1), Wout, bout.reshape(-1, 1))
    return out.reshape(_B, 1)


# speedup vs baseline: 1.1309x; 1.1309x over previous
import functools

import jax
import jax.numpy as jnp
from jax import lax
from jax.experimental import pallas as pl
from jax.experimental.pallas import tpu as pltpu
from jax.experimental.pallas import tpu_sc as plsc

_B, _NCAT, _NCONT, _V, _D = 16384, 26, 13, 100000, 32

_NROWS = _NCAT * _D
_NW = 32
_ROWS_PER_W = _NROWS // _NW
_CHUNK = 4096
_NCHUNK = _B // _CHUNK


def _gather_body(idx_hbm, tab_hbm, out_hbm, idx_v, row_v, out_a, out_b,
                 sem_a, sem_b):
    wid = lax.axis_index("s") * 2 + lax.axis_index("c")
    start = wid * _ROWS_PER_W
    end = start + _ROWS_PER_W
    mid = jnp.minimum((lax.div(start, _D) + 1) * _D, end)
    bufs = (out_a, out_b)
    sems = (sem_a, sem_b)

    def store_chunk(r, c, buf, sem):
        pltpu.async_copy(buf, out_hbm.at[r, pl.ds(c * _CHUNK, _CHUNK)], sem)

    def wait_store(buf, sem):
        pltpu.make_async_copy(buf, out_hbm.at[0, pl.ds(0, _CHUNK)],
                              sem).wait()

    def process_row(r, carry):
        pltpu.sync_copy(tab_hbm.at[r], row_v)
        for c in range(_NCHUNK):
            buf, sem = bufs[c % 2], sems[c % 2]
            wait_store(buf, sem)

            @plsc.parallel_loop(0, _CHUNK, step=16, unroll=16)
            def _g(k):
                i16 = idx_v[pl.ds(c * _CHUNK + k, 16)]
                buf[pl.ds(k, 16)] = plsc.load_gather(row_v, [i16])

            store_chunk(r, c, buf, sem)
        return carry

    pltpu.sync_copy(idx_hbm.at[lax.div(start, _D)], idx_v)
    store_chunk(end - 1, 2, out_a, sem_a)
    store_chunk(end - 1, 3, out_b, sem_b)
    lax.fori_loop(start, mid, process_row, 0)

    @pl.when(mid < end)
    def _second_field():
        pltpu.sync_copy(idx_hbm.at[lax.div(mid, _D)], idx_v)

    lax.fori_loop(mid, end, process_row, 0)
    wait_store(out_a, sem_a)
    wait_store(out_b, sem_b)


def _sc_gather(idxT, tabT):
    mesh = plsc.VectorSubcoreMesh(core_axis_name="c", subcore_axis_name="s")
    k = pl.kernel(
        _gather_body,
        mesh=mesh,
        out_type=jax.ShapeDtypeStruct((_NROWS, _B), jnp.float32),
        scratch_types=[
            pltpu.VMEM((_B,), jnp.int32),
            pltpu.VMEM((_V,), jnp.float32),
            pltpu.VMEM((_CHUNK,), jnp.float32),
            pltpu.VMEM((_CHUNK,), jnp.float32),
            pltpu.SemaphoreType.DMA,
            pltpu.SemaphoreType.DMA,
        ],
        compiler_params=pltpu.CompilerParams(needs_layout_passes=False),
    )
    return k(idxT, tabT)


_BT = 2048


def _mlp_body(cat_ref, x_ref, w1a_ref, w1c_ref, b1_ref, w2_ref, b2_ref,
              w3_ref, b3_ref, out_ref):
    x = x_ref[...]
    xc = jnp.where(jnp.isnan(x), 0.0, x)
    x1 = jnp.dot(w1a_ref[...], cat_ref[...], preferred_element_type=jnp.float32)
    x1 = x1 + jnp.dot(w1c_ref[...], xc, preferred_element_type=jnp.float32)
    x1 = jnp.maximum(x1 + b1_ref[...], 0.0)
    x2 = jnp.maximum(
        jnp.dot(w2_ref[...], x1, preferred_element_type=jnp.float32)
        + b2_ref[...], 0.0)
    out_ref[...] = (
        jnp.dot(w3_ref[...], x2, preferred_element_type=jnp.float32)
        + b3_ref[...])


def _mlp(catT, xT, w1a, w1c, b1, w2, b2, w3, b3):
    h1, h2 = w1a.shape[0], w2.shape[0]
    return pl.pallas_call(
        _mlp_body,
        grid=(_B // _BT,),
        in_specs=[
            pl.BlockSpec((_NROWS, _BT), lambda i: (0, i)),
            pl.BlockSpec((_NCONT, _BT), lambda i: (0, i)),
            pl.BlockSpec((h1, _NROWS), lambda i: (0, 0)),
            pl.BlockSpec((h1, _NCONT), lambda i: (0, 0)),
            pl.BlockSpec((h1, 1), lambda i: (0, 0)),
            pl.BlockSpec((h2, h1), lambda i: (0, 0)),
            pl.BlockSpec((h2, 1), lambda i: (0, 0)),
            pl.BlockSpec((1, h2), lambda i: (0, 0)),
            pl.BlockSpec((1, 1), lambda i: (0, 0)),
        ],
        out_specs=pl.BlockSpec((1, _BT), lambda i: (0, i)),
        out_shape=jax.ShapeDtypeStruct((1, _B), jnp.float32),
    )(catT, xT, w1a, w1c, b1, w2, b2, w3, b3)


def kernel(x_cat, x_cont, cat_tables, cont_tables, W1, b1, W2, b2, Wout, bout):
    idxT = x_cat.T
    tabT = cat_tables.transpose(0, 2, 1).reshape(_NROWS, _V)
    catT = _sc_gather(idxT, tabT)

    xT = x_cont.T
    w1a = W1[:, :_NROWS]
    w1c = jnp.einsum("id,jid->ji", cont_tables[:, 1, :],
                     W1[:, _NROWS:].reshape(-1, _NCONT, _D))
    out = _mlp(catT, xT, w1a, w1c, b1.reshape(-1, 1), W2,
               b2.reshape(-1, 1), Wout, bout.reshape(-1, 1))
    return out.reshape(_B, 1)
